# baseline (speedup 1.0000x reference)
import functools

import jax
import jax.numpy as jnp
from jax import lax
from jax.experimental import pallas as pl
from jax.experimental.pallas import tpu as pltpu
from jax.experimental.pallas import tpu_sc as plsc

NE_PAD = 64
LANES = 16


def _make_hist(B, L, num_cores, num_subcores):
    NW = num_cores * num_subcores
    rows = B // NW
    groups = rows // LANES
    gstride = LANES * 2 * L
    mesh = plsc.VectorSubcoreMesh(core_axis_name="c", subcore_axis_name="s")

    @functools.partial(
        pl.kernel,
        out_type=jax.ShapeDtypeStruct((B * NE_PAD,), jnp.float32),
        mesh=mesh,
        compiler_params=pltpu.CompilerParams(needs_layout_passes=False),
        scratch_types=[
            pltpu.VMEM((rows * 2 * L,), jnp.int32),
            pltpu.VMEM((rows * NE_PAD,), jnp.float32),
        ],
    )
    def hist(packed_hbm, w_hbm, packed_v, w_v):
        wid = lax.axis_index("s") * num_cores + lax.axis_index("c")
        base = wid * rows
        pltpu.sync_copy(packed_hbm.at[pl.ds(base * 2 * L, rows * 2 * L)],
                        packed_v)

        lane = lax.iota(jnp.int32, LANES)
        wbase0 = lane * NE_PAD
        zeros = jnp.zeros((LANES,), jnp.float32)

        def group_body(g, _):
            goff_w = g * (LANES * NE_PAD)
            for j in range(NE_PAD):
                w_v[pl.ds(goff_w + j * LANES, LANES)] = zeros

            wbase = wbase0 + goff_w
            goff_p = g * gstride
            for l in range(L):
                off = goff_p + l * (2 * LANES)
                c = packed_v[pl.ds(off, LANES)]
                mbits = packed_v[pl.ds(off + LANES, LANES)]
                m = plsc.bitcast(mbits, jnp.float32)
                plsc.addupdate_scatter(w_v, [wbase + c], m)
            return 0

        lax.fori_loop(0, groups, group_body, 0)
        pltpu.sync_copy(w_v, w_hbm.at[pl.ds(base * NE_PAD, rows * NE_PAD)])

    return hist


def _mm_body(w_ref, e_ref, o_ref):
    o_ref[...] = jnp.dot(w_ref[...], e_ref[...],
                         preferred_element_type=jnp.float32)


def kernel(cards, mask, embedding):
    B, L = cards.shape
    NE, D = embedding.shape
    info = plsc.get_sparse_core_info()

    cperm = cards.astype(jnp.int32).reshape(B // LANES, LANES, L)
    mperm = lax.bitcast_convert_type(mask, jnp.int32).reshape(
        B // LANES, LANES, L)
    packed = jnp.stack([cperm, mperm], axis=2)
    packed = packed.transpose(0, 3, 2, 1).reshape(-1)

    hist = _make_hist(B, L, info.num_cores, info.num_subcores)
    w = hist(packed).reshape(B, NE_PAD)

    epad = jnp.zeros((NE_PAD, D), jnp.float32).at[:NE].set(embedding)

    BM = 1024
    out = pl.pallas_call(
        _mm_body,
        grid=(B // BM,),
        in_specs=[
            pl.BlockSpec((BM, NE_PAD), lambda i: (i, 0)),
            pl.BlockSpec((NE_PAD, D), lambda i: (0, 0)),
        ],
        out_specs=pl.BlockSpec((BM, D), lambda i: (i, 0)),
        out_shape=jax.ShapeDtypeStruct((B, D), jnp.float32),
    )(w, epad)
    return out


# speedup vs baseline: 3.8893x; 3.8893x over previous
import functools

import jax
import jax.numpy as jnp
from jax import lax
from jax.experimental import pallas as pl
from jax.experimental.pallas import tpu as pltpu
from jax.experimental.pallas import tpu_sc as plsc

NE_PAD = 128
LANES = 16


def _make_hist(B, L, num_cores, num_subcores):
    NW = num_cores * num_subcores
    rows = B // NW
    groups = rows // LANES
    mesh = plsc.VectorSubcoreMesh(core_axis_name="c", subcore_axis_name="s")

    @functools.partial(
        pl.kernel,
        out_type=jax.ShapeDtypeStruct((B, NE_PAD), jnp.float32),
        mesh=mesh,
        compiler_params=pltpu.CompilerParams(needs_layout_passes=False),
        scratch_types=[
            pltpu.VMEM((2 * L, rows), jnp.int32),
            pltpu.VMEM((rows, NE_PAD), jnp.float32),
        ],
    )
    def hist(packed_hbm, w_hbm, packed_v, w_v):
        wid = lax.axis_index("s") * num_cores + lax.axis_index("c")
        base = wid * rows
        pltpu.sync_copy(packed_hbm.at[:, pl.ds(base, rows)], packed_v)

        lane = lax.iota(jnp.int32, LANES)
        zeros = jnp.zeros((LANES,), jnp.float32)

        def group_body(g, _):
            row0 = g * LANES
            for i in range(LANES):
                for j in range(NE_PAD // LANES):
                    w_v[row0 + i, pl.ds(j * LANES, LANES)] = zeros

            rowv = row0 + lane
            for l in range(L):
                c = packed_v[l, pl.ds(row0, LANES)]
                mbits = packed_v[L + l, pl.ds(row0, LANES)]
                m = plsc.bitcast(mbits, jnp.float32)
                plsc.addupdate_scatter(w_v, [rowv, c], m)
            return 0

        lax.fori_loop(0, groups, group_body, 0)
        pltpu.sync_copy(w_v, w_hbm.at[pl.ds(base, rows), :])

    return hist


def _mm_body(w_ref, e_ref, o_ref):
    o_ref[...] = jnp.dot(w_ref[...], e_ref[...],
                         preferred_element_type=jnp.float32)


def kernel(cards, mask, embedding):
    B, L = cards.shape
    NE, D = embedding.shape
    info = plsc.get_sparse_core_info()

    packed = jnp.concatenate(
        [cards.astype(jnp.int32).T,
         lax.bitcast_convert_type(mask, jnp.int32).T], axis=0)

    hist = _make_hist(B, L, info.num_cores, info.num_subcores)
    w = hist(packed)

    epad = jnp.zeros((NE_PAD, D), jnp.float32).at[:NE].set(embedding)

    BM = 1024
    out = pl.pallas_call(
        _mm_body,
        grid=(B // BM,),
        in_specs=[
            pl.BlockSpec((BM, NE_PAD), lambda i: (i, 0)),
            pl.BlockSpec((NE_PAD, D), lambda i: (0, 0)),
        ],
        out_specs=pl.BlockSpec((BM, D), lambda i: (i, 0)),
        out_shape=jax.ShapeDtypeStruct((B, D), jnp.float32),
    )(w, epad)
    return out


# speedup vs baseline: 4.6617x; 1.1986x over previous
import functools

import jax
import jax.numpy as jnp
from jax import lax
from jax.experimental import pallas as pl
from jax.experimental.pallas import tpu as pltpu
from jax.experimental.pallas import tpu_sc as plsc

NE_PAD = 128
LANES = 16


def _make_hist(B, L, num_cores, num_subcores):
    NW = num_cores * num_subcores
    rows = B // NW
    half = rows // 2
    hgroups = half // LANES
    mesh = plsc.VectorSubcoreMesh(core_axis_name="c", subcore_axis_name="s")

    @functools.partial(
        pl.kernel,
        out_type=jax.ShapeDtypeStruct((B, NE_PAD), jnp.float32),
        mesh=mesh,
        compiler_params=pltpu.CompilerParams(needs_layout_passes=False),
        scratch_types=[
            pltpu.VMEM((2 * L, rows), jnp.int32),
            pltpu.VMEM((rows, NE_PAD), jnp.float32),
            pltpu.SemaphoreType.DMA,
            pltpu.SemaphoreType.DMA,
            pltpu.SemaphoreType.DMA,
        ],
    )
    def hist(packed_hbm, w_hbm, packed_v, w_v, sem0, sem1, sem2):
        wid = lax.axis_index("s") * num_cores + lax.axis_index("c")
        base = wid * rows
        cp0 = pltpu.async_copy(packed_hbm.at[:, pl.ds(base, half)],
                               packed_v.at[:, pl.ds(0, half)], sem0)
        cp1 = pltpu.async_copy(packed_hbm.at[:, pl.ds(base + half, half)],
                               packed_v.at[:, pl.ds(half, half)], sem1)

        lane = lax.iota(jnp.int32, LANES)
        zeros = jnp.zeros((LANES,), jnp.float32)

        def make_group_body(row_off):
            def group_body(g):
                row0 = row_off + g * LANES
                for i in range(LANES):
                    for j in range(NE_PAD // LANES):
                        w_v[row0 + i, pl.ds(j * LANES, LANES)] = zeros

                rowv = row0 + lane
                for l in range(L):
                    c = packed_v[l, pl.ds(row0, LANES)]
                    mbits = packed_v[L + l, pl.ds(row0, LANES)]
                    m = plsc.bitcast(mbits, jnp.float32)
                    plsc.addupdate_scatter(w_v, [rowv, c], m)
            return group_body

        cp0.wait()
        plsc.parallel_loop(0, hgroups, 1, unroll=2)(make_group_body(0))
        cp1.wait()
        wb0 = pltpu.async_copy(w_v.at[pl.ds(0, half), :],
                               w_hbm.at[pl.ds(base, half), :], sem2)
        plsc.parallel_loop(0, hgroups, 1, unroll=2)(make_group_body(half))
        wb0.wait()
        pltpu.sync_copy(w_v.at[pl.ds(half, half), :],
                        w_hbm.at[pl.ds(base + half, half), :])

    return hist


def _mm_body(w_ref, e_ref, o_ref):
    o_ref[...] = jnp.dot(w_ref[...], e_ref[...],
                         preferred_element_type=jnp.float32)


def kernel(cards, mask, embedding):
    B, L = cards.shape
    NE, D = embedding.shape
    info = plsc.get_sparse_core_info()

    packed = jnp.concatenate(
        [cards.astype(jnp.int32).T,
         lax.bitcast_convert_type(mask, jnp.int32).T], axis=0)

    hist = _make_hist(B, L, info.num_cores, info.num_subcores)
    w = hist(packed)

    epad = jnp.zeros((NE_PAD, D), jnp.float32).at[:NE].set(embedding)

    BM = 2048
    out = pl.pallas_call(
        _mm_body,
        grid=(B // BM,),
        in_specs=[
            pl.BlockSpec((BM, NE_PAD), lambda i: (i, 0)),
            pl.BlockSpec((NE_PAD, D), lambda i: (0, 0)),
        ],
        out_specs=pl.BlockSpec((BM, D), lambda i: (i, 0)),
        out_shape=jax.ShapeDtypeStruct((B, D), jnp.float32),
    )(w, epad)
    return out


# speedup vs baseline: 4.9137x; 1.0541x over previous
import functools

import jax
import jax.numpy as jnp
from jax import lax
from jax.experimental import pallas as pl
from jax.experimental.pallas import tpu as pltpu
from jax.experimental.pallas import tpu_sc as plsc

NE_PAD = 128
LANES = 16


def _make_hist(B, L, num_cores, num_subcores):
    NW = num_cores * num_subcores
    rows = B // NW
    half = rows // 2
    hgroups = half // LANES
    mesh = plsc.VectorSubcoreMesh(core_axis_name="c", subcore_axis_name="s")

    @functools.partial(
        pl.kernel,
        out_type=jax.ShapeDtypeStruct((B, NE_PAD), jnp.float32),
        mesh=mesh,
        compiler_params=pltpu.CompilerParams(needs_layout_passes=False),
        scratch_types=[
            pltpu.VMEM((2 * L, rows), jnp.int32),
            pltpu.VMEM((rows, NE_PAD), jnp.float32),
            pltpu.SemaphoreType.DMA,
            pltpu.SemaphoreType.DMA,
            pltpu.SemaphoreType.DMA,
        ],
    )
    def hist(packed_hbm, w_hbm, packed_v, w_v, sem0, sem1, sem2):
        wid = lax.axis_index("s") * num_cores + lax.axis_index("c")
        base = wid * rows
        cp0 = pltpu.async_copy(packed_hbm.at[:, pl.ds(base, half)],
                               packed_v.at[:, pl.ds(0, half)], sem0)
        cp1 = pltpu.async_copy(packed_hbm.at[:, pl.ds(base + half, half)],
                               packed_v.at[:, pl.ds(half, half)], sem1)

        lane = lax.iota(jnp.int32, LANES)
        zeros = jnp.zeros((LANES,), jnp.float32)

        def make_group_body(row_off):
            def group_body(g):
                row0 = row_off + g * LANES
                for i in range(LANES):
                    for j in range(NE_PAD // LANES):
                        w_v[row0 + i, pl.ds(j * LANES, LANES)] = zeros

                rowv = row0 + lane
                for l in range(L):
                    c = packed_v[l, pl.ds(row0, LANES)]
                    mbits = packed_v[L + l, pl.ds(row0, LANES)]
                    m = plsc.bitcast(mbits, jnp.float32)
                    plsc.addupdate_scatter(w_v, [rowv, c], m)
            return group_body

        cp0.wait()
        plsc.parallel_loop(0, hgroups, 1, unroll=2)(make_group_body(0))
        cp1.wait()
        wb0 = pltpu.async_copy(w_v.at[pl.ds(0, half), :],
                               w_hbm.at[pl.ds(base, half), :], sem2)
        plsc.parallel_loop(0, hgroups, 1, unroll=2)(make_group_body(half))
        wb0.wait()
        pltpu.sync_copy(w_v.at[pl.ds(half, half), :],
                        w_hbm.at[pl.ds(base + half, half), :])

    return hist


def _mm_body(ne_pad, w_ref, e_ref, o_ref):
    ne = e_ref.shape[0]
    epad = jnp.pad(e_ref[...], ((0, ne_pad - ne), (0, 0)))
    o_ref[...] = jnp.dot(w_ref[...], epad,
                         preferred_element_type=jnp.float32)


def kernel(cards, mask, embedding):
    B, L = cards.shape
    NE, D = embedding.shape
    info = plsc.get_sparse_core_info()

    packed = jnp.concatenate(
        [cards.astype(jnp.int32).T,
         lax.bitcast_convert_type(mask, jnp.int32).T], axis=0)

    hist = _make_hist(B, L, info.num_cores, info.num_subcores)
    w = hist(packed)

    BM = 4096
    out = pl.pallas_call(
        functools.partial(_mm_body, NE_PAD),
        grid=(B // BM,),
        in_specs=[
            pl.BlockSpec((BM, NE_PAD), lambda i: (i, 0)),
            pl.BlockSpec((NE, D), lambda i: (0, 0)),
        ],
        out_specs=pl.BlockSpec((BM, D), lambda i: (i, 0)),
        out_shape=jax.ShapeDtypeStruct((B, D), jnp.float32),
    )(w, embedding)
    return out


# speedup vs baseline: 5.4184x; 1.1027x over previous
import functools

import jax
import jax.numpy as jnp
from jax import lax
from jax.experimental import pallas as pl
from jax.experimental.pallas import tpu as pltpu
from jax.experimental.pallas import tpu_sc as plsc

NE_PAD = 128
LANES = 16


def _make_hist(B, L, num_cores, num_subcores):
    NW = num_cores * num_subcores
    rows = B // NW
    half = rows // 2
    groups = rows // LANES
    mesh = plsc.VectorSubcoreMesh(core_axis_name="c", subcore_axis_name="s")

    @functools.partial(
        pl.kernel,
        out_type=jax.ShapeDtypeStruct((B, NE_PAD), jnp.float32),
        mesh=mesh,
        compiler_params=pltpu.CompilerParams(needs_layout_passes=False),
        scratch_types=[
            pltpu.VMEM((2 * L, rows), jnp.int32),
            pltpu.VMEM((rows, NE_PAD), jnp.float32),
            pltpu.SemaphoreType.DMA,
            pltpu.SemaphoreType.DMA,
            pltpu.SemaphoreType.DMA,
        ],
    )
    def hist(packed_hbm, w_hbm, packed_v, w_v, sem0, sem1, sem2):
        wid = lax.axis_index("s") * num_cores + lax.axis_index("c")
        base = wid * rows
        cp0 = pltpu.async_copy(packed_hbm.at[:, pl.ds(base, half)],
                               packed_v.at[:, pl.ds(0, half)], sem0)
        cp1 = pltpu.async_copy(packed_hbm.at[:, pl.ds(base + half, half)],
                               packed_v.at[:, pl.ds(half, half)], sem1)

        lane = lax.iota(jnp.int32, LANES)
        zeros = jnp.zeros((LANES,), jnp.float32)

        @plsc.parallel_loop(0, rows // 8)
        def _(z):
            for r in range(8):
                for j in range(NE_PAD // LANES):
                    w_v[z * 8 + r, pl.ds(j * LANES, LANES)] = zeros

        cp0.wait()
        cp1.wait()

        @plsc.parallel_loop(0, groups, unroll=2)
        def _(g):
            row0 = g * LANES
            rowv = row0 + lane
            for l in range(L):
                c = packed_v[l, pl.ds(row0, LANES)]
                mbits = packed_v[L + l, pl.ds(row0, LANES)]
                m = plsc.bitcast(mbits, jnp.float32)
                plsc.addupdate_scatter(w_v, [rowv, c], m)

        pltpu.sync_copy(w_v, w_hbm.at[pl.ds(base, rows), :])

    return hist


def _mm_body(ne_pad, w_ref, e_ref, o_ref):
    ne = e_ref.shape[0]
    epad = jnp.pad(e_ref[...], ((0, ne_pad - ne), (0, 0)))
    o_ref[...] = jnp.dot(w_ref[...], epad,
                         preferred_element_type=jnp.float32)


def kernel(cards, mask, embedding):
    B, L = cards.shape
    NE, D = embedding.shape
    info = plsc.get_sparse_core_info()

    packed = jnp.concatenate(
        [cards.astype(jnp.int32).T,
         lax.bitcast_convert_type(mask, jnp.int32).T], axis=0)

    hist = _make_hist(B, L, info.num_cores, info.num_subcores)
    w = hist(packed)

    BM = 4096
    out = pl.pallas_call(
        functools.partial(_mm_body, NE_PAD),
        grid=(B // BM,),
        in_specs=[
            pl.BlockSpec((BM, NE_PAD), lambda i: (i, 0)),
            pl.BlockSpec((NE, D), lambda i: (0, 0)),
        ],
        out_specs=pl.BlockSpec((BM, D), lambda i: (i, 0)),
        out_shape=jax.ShapeDtypeStruct((B, D), jnp.float32),
    )(w, embedding)
    return out
